# baseline (speedup 1.0000x reference)
import functools

import jax
import jax.numpy as jnp
from jax import lax
from jax.experimental import pallas as pl
from jax.experimental.pallas import tpu as pltpu
from jax.experimental.pallas import tpu_sc as plsc

BATCH = 16384
D = 16
ROWS_PER_BLK = 128 // D
NC = 2
NS = 16
NW = NC * NS
BPW = BATCH // NW
CHUNK = 256
NCH = BPW // CHUNK
NG = CHUNK // 16


def _body(user_hbm, item_i_hbm, item_j_hbm, utab_hbm, itab_hbm, out_hbm,
          uidx, iidx, jidx, ublk, iblk, jblk, urows, irows, jrows,
          pos_v, neg_v, sem):
    wid = lax.axis_index("s") * NC + lax.axis_index("c")
    base = wid * BPW

    pltpu.sync_copy(user_hbm.at[pl.ds(base, BPW)], uidx)
    pltpu.sync_copy(item_i_hbm.at[pl.ds(base, BPW)], iidx)
    pltpu.sync_copy(item_j_hbm.at[pl.ds(base, BPW)], jidx)

    def blkify(src, dst):
        def f(k, carry):
            v = src[pl.ds(k * 16, 16)]
            dst[pl.ds(k * 16, 16)] = lax.shift_right_logical(v, 3)
            return carry
        lax.fori_loop(0, BPW // 16, f, 0)

    blkify(uidx, ublk)
    blkify(iidx, iblk)
    blkify(jidx, jblk)

    tabs = ((utab_hbm, uidx, ublk, urows), (itab_hbm, iidx, iblk, irows),
            (itab_hbm, jidx, jblk, jrows))
    lane16 = lax.iota(jnp.int32, 16)

    for ch in range(NCH):
        off = ch * CHUNK
        copies = [
            pltpu.async_copy(tab.at[blk.at[pl.ds(off, CHUNK)]], rows, sem)
            for tab, _, blk, rows in tabs
        ]
        for c in copies:
            c.wait()

        def group(g, carry):
            r0 = off + g * 16
            rowv = lane16 + g * 16
            uv = uidx[pl.ds(r0, 16)]
            iv = iidx[pl.ds(r0, 16)]
            jv = jidx[pl.ds(r0, 16)]
            lu = (uv & 7) * D
            li = (iv & 7) * D
            lj = (jv & 7) * D
            accp = jnp.zeros((16,), jnp.float32)
            accn = jnp.zeros((16,), jnp.float32)
            for d in range(D):
                cu = plsc.load_gather(urows, [rowv, lu + d])
                ci = plsc.load_gather(irows, [rowv, li + d])
                cj = plsc.load_gather(jrows, [rowv, lj + d])
                accp = accp + cu * ci
                accn = accn + cu * cj
            pos_v[pl.ds(r0, 16)] = accp
            neg_v[pl.ds(r0, 16)] = accn
            return carry

        lax.fori_loop(0, NG, group, 0)

    pltpu.sync_copy(pos_v, out_hbm.at[pl.ds(base, BPW)])
    pltpu.sync_copy(neg_v, out_hbm.at[pl.ds(BATCH + base, BPW)])


def kernel(user, item_i, item_j, user_table, item_table):
    user = user.astype(jnp.int32)
    item_i = item_i.astype(jnp.int32)
    item_j = item_j.astype(jnp.int32)
    utab = user_table.reshape(-1, 128)
    itab = item_table.reshape(-1, 128)
    mesh = plsc.VectorSubcoreMesh(core_axis_name="c", subcore_axis_name="s")
    run = functools.partial(
        pl.kernel,
        mesh=mesh,
        compiler_params=pltpu.CompilerParams(needs_layout_passes=False),
        out_type=jax.ShapeDtypeStruct((2 * BATCH,), jnp.float32),
        scratch_types=[
            pltpu.VMEM((BPW,), jnp.int32),
            pltpu.VMEM((BPW,), jnp.int32),
            pltpu.VMEM((BPW,), jnp.int32),
            pltpu.VMEM((BPW,), jnp.int32),
            pltpu.VMEM((BPW,), jnp.int32),
            pltpu.VMEM((BPW,), jnp.int32),
            pltpu.VMEM((CHUNK, 128), jnp.float32),
            pltpu.VMEM((CHUNK, 128), jnp.float32),
            pltpu.VMEM((CHUNK, 128), jnp.float32),
            pltpu.VMEM((BPW,), jnp.float32),
            pltpu.VMEM((BPW,), jnp.float32),
            pltpu.SemaphoreType.DMA,
        ],
    )(_body)
    return run(user, item_i, item_j, utab, itab)


# speedup vs baseline: 1.0025x; 1.0025x over previous
import functools

import jax
import jax.numpy as jnp
from jax import lax
from jax.experimental import pallas as pl
from jax.experimental.pallas import tpu as pltpu
from jax.experimental.pallas import tpu_sc as plsc

BATCH = 16384
D = 16
ROWS_PER_BLK = 128 // D
NC = 2
NS = 16
NW = NC * NS
BPW = BATCH // NW
CHUNK = 256
NCH = BPW // CHUNK


def _body(user_hbm, item_i_hbm, item_j_hbm, utab_hbm, itab_hbm, out_hbm,
          uidx, iidx, jidx, ublk, iblk, jblk, urows, irows, jrows,
          pos_v, neg_v, sem):
    wid = lax.axis_index("s") * NC + lax.axis_index("c")
    base = wid * BPW

    pltpu.sync_copy(user_hbm.at[pl.ds(base, BPW)], uidx)
    pltpu.sync_copy(item_i_hbm.at[pl.ds(base, BPW)], iidx)
    pltpu.sync_copy(item_j_hbm.at[pl.ds(base, BPW)], jidx)

    for src, dst in ((uidx, ublk), (iidx, iblk), (jidx, jblk)):
        for k in range(BPW // 16):
            dst[pl.ds(k * 16, 16)] = lax.shift_right_logical(
                src[pl.ds(k * 16, 16)], 3)

    tabs = ((utab_hbm, uidx, ublk, urows), (itab_hbm, iidx, iblk, irows),
            (itab_hbm, jidx, jblk, jrows))
    lane16 = lax.iota(jnp.int32, 16)

    for ch in range(NCH):
        off = ch * CHUNK
        copies = [
            pltpu.async_copy(tab.at[blk.at[pl.ds(off, CHUNK)]], rows, sem)
            for tab, _, blk, rows in tabs
        ]
        for c in copies:
            c.wait()

        @plsc.parallel_loop(off, off + CHUNK, step=16, carry=jnp.int32(0))
        def _group(r0, carry):
            rowv = lane16 + (r0 - off)
            uv = uidx[pl.ds(r0, 16)]
            iv = iidx[pl.ds(r0, 16)]
            jv = jidx[pl.ds(r0, 16)]
            lu = (uv & 7) * D
            li = (iv & 7) * D
            lj = (jv & 7) * D
            accp = jnp.zeros((16,), jnp.float32)
            accn = jnp.zeros((16,), jnp.float32)
            for d in range(D):
                cu = plsc.load_gather(urows, [rowv, lu + d])
                ci = plsc.load_gather(irows, [rowv, li + d])
                cj = plsc.load_gather(jrows, [rowv, lj + d])
                accp = accp + cu * ci
                accn = accn + cu * cj
            pos_v[pl.ds(r0, 16)] = accp
            neg_v[pl.ds(r0, 16)] = accn
            return carry

    pltpu.sync_copy(pos_v, out_hbm.at[pl.ds(base, BPW)])
    pltpu.sync_copy(neg_v, out_hbm.at[pl.ds(BATCH + base, BPW)])


def kernel(user, item_i, item_j, user_table, item_table):
    user = user.astype(jnp.int32)
    item_i = item_i.astype(jnp.int32)
    item_j = item_j.astype(jnp.int32)
    utab = user_table.reshape(-1, 128)
    itab = item_table.reshape(-1, 128)
    mesh = plsc.VectorSubcoreMesh(core_axis_name="c", subcore_axis_name="s")
    run = functools.partial(
        pl.kernel,
        mesh=mesh,
        compiler_params=pltpu.CompilerParams(needs_layout_passes=False),
        out_type=jax.ShapeDtypeStruct((2 * BATCH,), jnp.float32),
        scratch_types=[
            pltpu.VMEM((BPW,), jnp.int32),
            pltpu.VMEM((BPW,), jnp.int32),
            pltpu.VMEM((BPW,), jnp.int32),
            pltpu.VMEM((BPW,), jnp.int32),
            pltpu.VMEM((BPW,), jnp.int32),
            pltpu.VMEM((BPW,), jnp.int32),
            pltpu.VMEM((CHUNK, 128), jnp.float32),
            pltpu.VMEM((CHUNK, 128), jnp.float32),
            pltpu.VMEM((CHUNK, 128), jnp.float32),
            pltpu.VMEM((BPW,), jnp.float32),
            pltpu.VMEM((BPW,), jnp.float32),
            pltpu.SemaphoreType.DMA,
        ],
    )(_body)
    return run(user, item_i, item_j, utab, itab)
